# baseline (device time: 29358 ns/iter reference)
import jax
import jax.numpy as jnp
from jax import lax
from jax.experimental import pallas as pl
from jax.experimental.pallas import tpu as pltpu

T = 256
D = 512
V = 4096
Q = T // 4
R = Q // 2
PV = V + 128


def kernel(x, W):
    assert x.shape == (T, D), x.shape
    assert W.shape == (D, V), W.shape

    def body(x_hbm, w_hbm, out_ref, x_ref, w_ref,
             xsend, xrecv, yfrecv, zfrecv, yrrecv, zrrecv,
             eloc_ref, sloc_ref, in_sems,
             xs, xr, yfs, yfr, zfs, zfr, yrs, yrr, zrs, zrr):
        my_x = lax.axis_index("x")
        my_y = lax.axis_index("y")
        my_z = lax.axis_index("z")
        zq = lax.rem(my_z, 2)
        x_partner = (1 - my_x, my_y, my_z)
        y_partner = (my_x, 1 - my_y, my_z)
        z_partner = (my_x, my_y, my_z + 1 - 2 * zq)
        loc_off = my_x * V
        rem_off = (1 - my_x) * V
        base_p = my_y * 128 + zq * Q
        base_y = (1 - my_y) * 128 + zq * Q
        base_z = my_y * 128 + (1 - zq) * Q
        base_d = (1 - my_y) * 128 + (1 - zq) * Q

        w_cp = pltpu.make_async_copy(w_hbm, w_ref, in_sems.at[0])
        x_cp = pltpu.make_async_copy(x_hbm, x_ref, in_sems.at[1])
        w_cp.start()
        x_cp.start()

        barrier = pltpu.get_barrier_semaphore()
        for nbr in (x_partner, y_partner, z_partner):
            pl.semaphore_signal(barrier, inc=1, device_id=nbr,
                                device_id_type=pl.DeviceIdType.MESH)
        pl.semaphore_wait(barrier, 3)

        w_cp.wait()
        x_cp.wait()
        w_bf = w_ref[:, :].astype(jnp.bfloat16)

        def local_chunk(base, i):
            rows = pl.ds(base + i * R, R)
            logits = jnp.dot(x_ref[rows, :].astype(jnp.bfloat16), w_bf,
                             preferred_element_type=jnp.float32)
            e = jnp.exp(logits)
            s = jnp.sum(e, axis=-1, keepdims=True)
            eloc_ref[rows, :] = e
            sloc_ref[rows, :] = s
            return e, s

        def rdma(src, dst, ssem, rsem, dev):
            return pltpu.make_async_remote_copy(
                src_ref=src, dst_ref=dst, send_sem=ssem, recv_sem=rsem,
                device_id=dev, device_id_type=pl.DeviceIdType.MESH)

        x_rdmas = []
        for c in range(2):
            e, s = local_chunk(base_p, c)
            xsend[c] = jnp.concatenate(
                [e.astype(jnp.bfloat16),
                 jnp.broadcast_to(s.astype(jnp.bfloat16), (R, 128))],
                axis=1)
            rd = rdma(xsend.at[c], xrecv.at[c], xs.at[c], xr.at[c], x_partner)
            rd.start()
            x_rdmas.append(rd)

        yfwds, zfwds = [], []

        def fwd_chunk(c):
            x_rdmas[c].wait_recv()
            yf = rdma(xrecv.at[c], yfrecv.at[c], yfs.at[c], yfr.at[c],
                      y_partner)
            zf = rdma(xrecv.at[c], zfrecv.at[c], zfs.at[c], zfr.at[c],
                      z_partner)
            yf.start()
            zf.start()
            yfwds.append(yf)
            zfwds.append(zf)

        fwd_chunk(0)

        for base in (base_y, base_z, base_d):
            for c in range(2):
                local_chunk(base, c)

        fwd_chunk(1)

        zfwds[0].wait_recv()
        yrel = rdma(zfrecv.at[0], yrrecv, yrs, yrr, y_partner)
        yrel.start()
        yfwds[1].wait_recv()
        zrel = rdma(yfrecv.at[1], zrrecv, zrs, zrr, z_partner)
        zrel.start()

        def finish(base, c, blk):
            rows = pl.ds(base + c * R, R)
            e_rem = blk[:, :V].astype(jnp.float32)
            s_rem = blk[:, V:V + 128].astype(jnp.float32)[:, 0:1]
            inv = 1.0 / (sloc_ref[rows, :] + s_rem)
            out_ref[rows, pl.ds(loc_off, V)] = eloc_ref[rows, :] * inv
            out_ref[rows, pl.ds(rem_off, V)] = e_rem * inv

        finish(base_p, 0, xrecv[0])
        finish(base_p, 1, xrecv[1])
        yfwds[0].wait_recv()
        finish(base_y, 0, yfrecv[0])
        finish(base_y, 1, yfrecv[1])
        finish(base_z, 0, zfrecv[0])
        zfwds[1].wait_recv()
        finish(base_z, 1, zfrecv[1])
        yrel.wait_recv()
        finish(base_d, 0, yrrecv[:, :])
        zrel.wait_recv()
        finish(base_d, 1, zrrecv[:, :])

        for rd in x_rdmas + yfwds + zfwds + [yrel, zrel]:
            rd.wait_send()

    return pl.pallas_call(
        body,
        out_shape=jax.ShapeDtypeStruct((T, 2 * V), jnp.float32),
        in_specs=[
            pl.BlockSpec(memory_space=pltpu.MemorySpace.HBM),
            pl.BlockSpec(memory_space=pltpu.MemorySpace.HBM),
        ],
        out_specs=pl.BlockSpec(memory_space=pltpu.VMEM),
        scratch_shapes=[
            pltpu.VMEM((T, D), jnp.float32),
            pltpu.VMEM((D, V), jnp.float32),
            pltpu.VMEM((2, R, PV), jnp.bfloat16),
            pltpu.VMEM((2, R, PV), jnp.bfloat16),
            pltpu.VMEM((2, R, PV), jnp.bfloat16),
            pltpu.VMEM((2, R, PV), jnp.bfloat16),
            pltpu.VMEM((R, PV), jnp.bfloat16),
            pltpu.VMEM((R, PV), jnp.bfloat16),
            pltpu.VMEM((T, V), jnp.float32),
            pltpu.VMEM((T, 1), jnp.float32),
            pltpu.SemaphoreType.DMA((2,)),
            pltpu.SemaphoreType.DMA((2,)),
            pltpu.SemaphoreType.DMA((2,)),
            pltpu.SemaphoreType.DMA((2,)),
            pltpu.SemaphoreType.DMA((2,)),
            pltpu.SemaphoreType.DMA((2,)),
            pltpu.SemaphoreType.DMA((2,)),
            pltpu.SemaphoreType.DMA,
            pltpu.SemaphoreType.DMA,
            pltpu.SemaphoreType.DMA,
            pltpu.SemaphoreType.DMA,
        ],
        compiler_params=pltpu.CompilerParams(collective_id=0),
    )(
        pltpu.with_memory_space_constraint(x, pltpu.MemorySpace.HBM),
        pltpu.with_memory_space_constraint(W, pltpu.MemorySpace.HBM),
    )
